# baseline (speedup 1.0000x reference)
import functools

import jax
import jax.numpy as jnp
from jax import lax
from jax.experimental import pallas as pl
from jax.experimental.pallas import tpu as pltpu
from jax.experimental.pallas import tpu_sc as plsc
import numpy as np

B = 4096
L = 200
E = 32
N = B * L
NCHUNK = 2
NK = N // NCHUNK
BK = B // NCHUNK

NC = 2
NS = 16
NW = NC * NS
PER_W = NK // NW
G = 128
S = 256
GPS = S // G
NSTAGES = PER_W // S
NPAIRS = NSTAGES // 2
ROWS_W = PER_W // G

C0, C1, C2, C3, C4 = (0.9999590188675769, -4.932735512906164, 4.041964638154526,
                      -1.2873554659573256, 0.1782067264910494)
S0, S1, S2, S3 = (3.1392768843462933, -5.136388565767432, 2.434666512020243,
                  -0.43779898378705956)

_MESH = plsc.VectorSubcoreMesh(core_axis_name="c", subcore_axis_name="s")


@functools.partial(
    pl.kernel,
    out_type=(jax.ShapeDtypeStruct((4, NK // G * 8, G), jnp.float32),
              jax.ShapeDtypeStruct((4, NK // G * 8, G), jnp.float32)),
    mesh=_MESH,
    compiler_params=pltpu.CompilerParams(needs_layout_passes=False,
                                         use_tc_tiling_on_sc=False),
    scratch_types=[
        pltpu.VMEM((ROWS_W, G + 16), jnp.int32),
        pltpu.VMEM((GPS, G), jnp.int32),
        pltpu.VMEM((GPS, G), jnp.int32),
        pltpu.VMEM((S, E), jnp.float32),
        pltpu.VMEM((S, E), jnp.float32),
        pltpu.VMEM((4 * GPS * 8, G + 1), jnp.float32),
        pltpu.VMEM((4 * GPS * 8, G + 1), jnp.float32),
        pltpu.VMEM((4 * GPS * 8, G + 1), jnp.float32),
        pltpu.VMEM((4 * GPS * 8, G + 1), jnp.float32),
        pltpu.SemaphoreType.DMA,
        pltpu.SemaphoreType.DMA,
        pltpu.SemaphoreType.DMA,
        pltpu.SemaphoreType.DMA,
    ],
)
def _sc_encode(enc_hbm, table_hbm, outr_hbm, outi_hbm,
               enc_v, dec0, dec1, rows0, rows1,
               outr0, outr1, outi0, outi1,
               gsem0, gsem1, osem0, osem1):
    wid = lax.axis_index("s") * NC + lax.axis_index("c")
    decs = (dec0, dec1)
    rows = (rows0, rows1)
    outr = (outr0, outr1)
    outi = (outi0, outi1)
    gsems = (gsem0, gsem1)
    osems = (osem0, osem1)

    pltpu.sync_copy(enc_hbm.at[pl.ds(wid * ROWS_W, ROWS_W)],
                    enc_v.at[pl.ds(0, ROWS_W), pl.ds(0, G)])

    def decode(s, b):
        for g in range(GPS):
            for v in range(G // 16):
                x = enc_v[s * GPS + g, pl.ds(v * 16, 16)]
                decs[b][g, pl.ds(v * 16, 16)] = lax.shift_right_logical(x, 1)

    def fire_gather(b):
        for g in range(GPS):
            pltpu.async_copy(table_hbm.at[decs[b].at[g]],
                             rows[b].at[pl.ds(g * G, G)], gsems[b])

    def drain_gather(b):
        pltpu.make_async_copy(table_hbm.at[pl.ds(0, S)], rows[b], gsems[b]).wait()

    def fire_out(s, b):
        r0 = (wid * ROWS_W + s * GPS) * 8
        for rt in range(4):
            pltpu.async_copy(outr[b].at[pl.ds(rt * GPS * 8, GPS * 8), pl.ds(0, G)],
                             outr_hbm.at[rt, pl.ds(r0, GPS * 8)], osems[b])
            pltpu.async_copy(outi[b].at[pl.ds(rt * GPS * 8, GPS * 8), pl.ds(0, G)],
                             outi_hbm.at[rt, pl.ds(r0, GPS * 8)], osems[b])

    def drain_out(b):
        pltpu.make_async_copy(outr[b].at[pl.ds(0, 4 * GPS * 8), pl.ds(0, G)],
                              outr_hbm.at[pl.ds(0, 4), pl.ds(0, GPS * 8)],
                              osems[b]).wait()
        pltpu.make_async_copy(outi[b].at[pl.ds(0, 4 * GPS * 8), pl.ds(0, G)],
                              outi_hbm.at[pl.ds(0, 4), pl.ds(0, GPS * 8)],
                              osems[b]).wait()

    def compute_stage(s, b):
        iota = lax.iota(jnp.int32, 16)
        rb0 = lax.shift_right_logical(iota, 3) * (GPS * 8) + (iota & 7)
        rb1 = lax.shift_right_logical(iota + 16, 3) * (GPS * 8) + ((iota + 16) & 7)

        def grp_body(g, carry):
            pb = g * 16
            ct = lax.shift_right_logical(pb, 7)
            col0 = pb & 127
            enc = enc_v[s * GPS + ct, pl.ds(col0, 16)]
            ampv = 1.0 - (enc & 1).astype(jnp.float32)
            row0 = rb0 + jnp.full((16,), ct * 8, dtype=jnp.int32)
            row1 = rb1 + jnp.full((16,), ct * 8, dtype=jnp.int32)
            for j in range(16):
                p = pb + j
                amp = jnp.full((16,), ampv[j], dtype=jnp.float32)
                colv = jnp.full((16,), col0 + j, dtype=jnp.int32)
                for rowv, half in ((row0, 0), (row1, 1)):
                    x = rows[b][p, pl.ds(16 * half, 16)]
                    ex = jnp.exp(x + x)
                    t = 1.0 - 2.0 / (ex + 1.0)
                    z = t * t
                    cv = C0 + z * (C1 + z * (C2 + z * (C3 + z * C4)))
                    sv = t * (S0 + z * (S1 + z * (S2 + z * S3)))
                    plsc.store_scatter(outr[b], [rowv, colv], cv * amp)
                    plsc.store_scatter(outi[b], [rowv, colv], sv * amp)
            return carry
        lax.fori_loop(0, S // 16, grp_body, 0)

    decode(0, 0)
    fire_gather(0)
    decode(1, 1)
    fire_gather(1)

    for b in (0, 1):
        drain_gather(b)
        compute_stage(b, b)
        fire_out(b, b)
        decode(b + 2, b)
        fire_gather(b)

    def pair_body(p, carry):
        for b in (0, 1):
            s = 2 * p + b
            drain_gather(b)
            drain_out(b)
            compute_stage(s, b)
            fire_out(s, b)
            decode(s + 2, b)
            fire_gather(b)
        return carry
    lax.fori_loop(1, NPAIRS - 1, pair_body, 0)

    for b in (0, 1):
        s = 2 * (NPAIRS - 1) + b
        drain_gather(b)
        drain_out(b)
        compute_stage(s, b)
        fire_out(s, b)

    drain_out(0)
    drain_out(1)


def kernel(input_ids, mask, table):
    enc = (input_ids.astype(jnp.int32) * 2 + mask.astype(jnp.int32))
    enc2d = enc.reshape(N // G, G)
    chunks = []
    for k in range(NCHUNK):
        enc_k = lax.slice(enc2d, (k * (NK // G), 0), ((k + 1) * (NK // G), G))
        r4, i4 = _sc_encode(enc_k, table)
        r4 = r4.reshape(4, NK // G, 8, G)
        i4 = i4.reshape(4, NK // G, 8, G)
        re = r4.transpose(0, 2, 1, 3).reshape(E, NK).T.reshape(BK, L, E)
        im = i4.transpose(0, 2, 1, 3).reshape(E, NK).T.reshape(BK, L, E)
        chunks.append(lax.complex(re, im))
    return jnp.concatenate(chunks, axis=0)


# speedup vs baseline: 1.0003x; 1.0003x over previous
import functools

import jax
import jax.numpy as jnp
from jax import lax
from jax.experimental import pallas as pl
from jax.experimental.pallas import tpu as pltpu
from jax.experimental.pallas import tpu_sc as plsc
import numpy as np

B = 4096
L = 200
E = 32
N = B * L
NCHUNK = 2
NK = N // NCHUNK
BK = B // NCHUNK

NC = 2
NS = 16
NW = NC * NS
PER_W = NK // NW
G = 128
S = 256
GPS = S // G
NSTAGES = PER_W // S
NPAIRS = NSTAGES // 2
ROWS_W = PER_W // G

C0, C1, C2, C3, C4 = (0.9999590188675769, -4.932735512906164, 4.041964638154526,
                      -1.2873554659573256, 0.1782067264910494)
S0, S1, S2, S3 = (3.1392768843462933, -5.136388565767432, 2.434666512020243,
                  -0.43779898378705956)

_MESH = plsc.VectorSubcoreMesh(core_axis_name="c", subcore_axis_name="s")


@functools.partial(
    pl.kernel,
    out_type=(jax.ShapeDtypeStruct((4, NK // G * 8, G), jnp.float32),
              jax.ShapeDtypeStruct((4, NK // G * 8, G), jnp.float32)),
    mesh=_MESH,
    compiler_params=pltpu.CompilerParams(needs_layout_passes=False,
                                         use_tc_tiling_on_sc=False),
    scratch_types=[
        pltpu.VMEM((ROWS_W, G + 16), jnp.int32),
        pltpu.VMEM((GPS, G), jnp.int32),
        pltpu.VMEM((GPS, G), jnp.int32),
        pltpu.VMEM((S, E), jnp.float32),
        pltpu.VMEM((S, E), jnp.float32),
        pltpu.VMEM((4 * GPS * 8, G + 1), jnp.float32),
        pltpu.VMEM((4 * GPS * 8, G + 1), jnp.float32),
        pltpu.VMEM((4 * GPS * 8, G + 1), jnp.float32),
        pltpu.VMEM((4 * GPS * 8, G + 1), jnp.float32),
        pltpu.SemaphoreType.DMA,
        pltpu.SemaphoreType.DMA,
        pltpu.SemaphoreType.DMA,
        pltpu.SemaphoreType.DMA,
    ],
)
def _sc_encode(enc_hbm, table_hbm, outr_hbm, outi_hbm,
               enc_v, dec0, dec1, rows0, rows1,
               outr0, outr1, outi0, outi1,
               gsem0, gsem1, osem0, osem1):
    wid = lax.axis_index("s") * NC + lax.axis_index("c")
    decs = (dec0, dec1)
    rows = (rows0, rows1)
    outr = (outr0, outr1)
    outi = (outi0, outi1)
    gsems = (gsem0, gsem1)
    osems = (osem0, osem1)

    pltpu.sync_copy(enc_hbm.at[pl.ds(wid * ROWS_W, ROWS_W)],
                    enc_v.at[pl.ds(0, ROWS_W), pl.ds(0, G)])

    def decode(s, b):
        for g in range(GPS):
            for v in range(G // 16):
                x = enc_v[s * GPS + g, pl.ds(v * 16, 16)]
                decs[b][g, pl.ds(v * 16, 16)] = lax.shift_right_logical(x, 1)

    def fire_gather(b):
        for g in range(GPS):
            pltpu.async_copy(table_hbm.at[decs[b].at[g]],
                             rows[b].at[pl.ds(g * G, G)], gsems[b])

    def drain_gather(b):
        pltpu.make_async_copy(table_hbm.at[pl.ds(0, S)], rows[b], gsems[b]).wait()

    def fire_out(s, b):
        r0 = (wid * ROWS_W + s * GPS) * 8
        for rt in range(4):
            pltpu.async_copy(outr[b].at[pl.ds(rt * GPS * 8, GPS * 8), pl.ds(0, G)],
                             outr_hbm.at[rt, pl.ds(r0, GPS * 8)], osems[b])
            pltpu.async_copy(outi[b].at[pl.ds(rt * GPS * 8, GPS * 8), pl.ds(0, G)],
                             outi_hbm.at[rt, pl.ds(r0, GPS * 8)], osems[b])

    def drain_out(b):
        pltpu.make_async_copy(outr[b].at[pl.ds(0, 4 * GPS * 8), pl.ds(0, G)],
                              outr_hbm.at[pl.ds(0, 4), pl.ds(0, GPS * 8)],
                              osems[b]).wait()
        pltpu.make_async_copy(outi[b].at[pl.ds(0, 4 * GPS * 8), pl.ds(0, G)],
                              outi_hbm.at[pl.ds(0, 4), pl.ds(0, GPS * 8)],
                              osems[b]).wait()

    def compute_stage(s, b):
        iota = lax.iota(jnp.int32, 16)
        rb0 = lax.shift_right_logical(iota, 3) * (GPS * 8) + (iota & 7)
        rb1 = lax.shift_right_logical(iota + 16, 3) * (GPS * 8) + ((iota + 16) & 7)

        @plsc.parallel_loop(0, S // 16)
        def grp_body(g):
            pb = g * 16
            ct = lax.shift_right_logical(pb, 7)
            col0 = pb & 127
            enc = enc_v[s * GPS + ct, pl.ds(col0, 16)]
            ampv = 1.0 - (enc & 1).astype(jnp.float32)
            row0 = rb0 + jnp.full((16,), ct * 8, dtype=jnp.int32)
            row1 = rb1 + jnp.full((16,), ct * 8, dtype=jnp.int32)
            for j in range(16):
                p = pb + j
                amp = jnp.full((16,), ampv[j], dtype=jnp.float32)
                colv = jnp.full((16,), col0 + j, dtype=jnp.int32)
                for rowv, half in ((row0, 0), (row1, 1)):
                    x = rows[b][p, pl.ds(16 * half, 16)]
                    ex = jnp.exp(x + x)
                    t = 1.0 - 2.0 / (ex + 1.0)
                    z = t * t
                    cv = C0 + z * (C1 + z * (C2 + z * (C3 + z * C4)))
                    sv = t * (S0 + z * (S1 + z * (S2 + z * S3)))
                    plsc.store_scatter(outr[b], [rowv, colv], cv * amp)
                    plsc.store_scatter(outi[b], [rowv, colv], sv * amp)

    decode(0, 0)
    fire_gather(0)
    decode(1, 1)
    fire_gather(1)

    for b in (0, 1):
        drain_gather(b)
        compute_stage(b, b)
        fire_out(b, b)
        decode(b + 2, b)
        fire_gather(b)

    def pair_body(p, carry):
        for b in (0, 1):
            s = 2 * p + b
            drain_gather(b)
            drain_out(b)
            compute_stage(s, b)
            fire_out(s, b)
            decode(s + 2, b)
            fire_gather(b)
        return carry
    lax.fori_loop(1, NPAIRS - 1, pair_body, 0)

    for b in (0, 1):
        s = 2 * (NPAIRS - 1) + b
        drain_gather(b)
        drain_out(b)
        compute_stage(s, b)
        fire_out(s, b)

    drain_out(0)
    drain_out(1)


def kernel(input_ids, mask, table):
    enc = (input_ids.astype(jnp.int32) * 2 + mask.astype(jnp.int32))
    enc2d = enc.reshape(N // G, G)
    chunks = []
    for k in range(NCHUNK):
        enc_k = lax.slice(enc2d, (k * (NK // G), 0), ((k + 1) * (NK // G), G))
        r4, i4 = _sc_encode(enc_k, table)
        r4 = r4.reshape(4, NK // G, 8, G)
        i4 = i4.reshape(4, NK // G, 8, G)
        re = r4.transpose(0, 2, 1, 3).reshape(E, NK).T.reshape(BK, L, E)
        im = i4.transpose(0, 2, 1, 3).reshape(E, NK).T.reshape(BK, L, E)
        chunks.append(lax.complex(re, im))
    return jnp.concatenate(chunks, axis=0)


# speedup vs baseline: 1.0597x; 1.0594x over previous
import functools

import jax
import jax.numpy as jnp
from jax import lax
from jax.experimental import pallas as pl
from jax.experimental.pallas import tpu as pltpu
from jax.experimental.pallas import tpu_sc as plsc

B = 4096
L = 200
E = 32
N = B * L

NC = 2
NS = 16
NW = NC * NS
PER_W = N // NW
G = 128
S = 512
GPS = S // G
NSTAGES = PER_W // S
NPAIRS = NSTAGES // 2

C0, C1, C2, C3, C4 = (0.9999590188675769, -4.932735512906164, 4.041964638154526,
                      -1.2873554659573256, 0.1782067264910494)
S0, S1, S2, S3 = (3.1392768843462933, -5.136388565767432, 2.434666512020243,
                  -0.43779898378705956)

_MESH = plsc.VectorSubcoreMesh(core_axis_name="c", subcore_axis_name="s")


@functools.partial(
    pl.kernel,
    out_type=jax.ShapeDtypeStruct((N, 2 * E), jnp.bfloat16),
    mesh=_MESH,
    compiler_params=pltpu.CompilerParams(needs_layout_passes=False,
                                         use_tc_tiling_on_sc=False),
    scratch_types=[
        pltpu.VMEM((PER_W // G, G), jnp.int32),
        pltpu.VMEM((PER_W + 16,), jnp.float32),
        pltpu.VMEM((S, E), jnp.float32),
        pltpu.VMEM((S, E), jnp.float32),
        pltpu.VMEM((S, 2 * E), jnp.bfloat16),
        pltpu.VMEM((S, 2 * E), jnp.bfloat16),
        pltpu.SemaphoreType.DMA,
        pltpu.SemaphoreType.DMA,
        pltpu.SemaphoreType.DMA,
        pltpu.SemaphoreType.DMA,
    ],
)
def _sc_encode(ids_hbm, maskf_hbm, table_hbm, out_hbm,
               idx_v, msk_v, rows0, rows1, out0, out1,
               gsem0, gsem1, osem0, osem1):
    wid = lax.axis_index("s") * NC + lax.axis_index("c")
    rows = (rows0, rows1)
    outs = (out0, out1)
    gsems = (gsem0, gsem1)
    osems = (osem0, osem1)

    pltpu.sync_copy(ids_hbm.at[pl.ds(wid * (PER_W // G), PER_W // G)], idx_v)
    pltpu.sync_copy(maskf_hbm.at[pl.ds(wid * PER_W, PER_W)],
                    msk_v.at[pl.ds(0, PER_W)])

    def fire_gather(s, buf, sem):
        for g in range(GPS):
            pltpu.async_copy(table_hbm.at[idx_v.at[s * GPS + g]],
                             buf.at[pl.ds(g * G, G)], sem)

    def drain_gather(buf, sem):
        pltpu.make_async_copy(table_hbm.at[pl.ds(0, S)], buf, sem).wait()

    def fire_out(s, buf, sem):
        dst = out_hbm.at[pl.ds(wid * PER_W + s * S, S)]
        pltpu.async_copy(buf, dst, sem)

    def drain_out(buf, sem):
        pltpu.make_async_copy(buf, out_hbm.at[pl.ds(0, S)], sem).wait()

    def compute_stage(s, buf, obuf):
        def row_body(r, carry):
            m = msk_v[pl.ds(s * S + r, 16)][0]
            amp = jnp.full((16,), 1.0 - m, dtype=jnp.float32)
            for half in (0, 1):
                x = buf[r, pl.ds(16 * half, 16)]
                e = jnp.exp(x + x)
                t = 1.0 - 2.0 / (e + 1.0)
                z = t * t
                cv = C0 + z * (C1 + z * (C2 + z * (C3 + z * C4)))
                sv = t * (S0 + z * (S1 + z * (S2 + z * S3)))
                obuf[r, pl.ds(32 * half, 32)] = plsc.pack(
                    cv * amp, sv * amp, format=plsc.PackFormat.INTERLEAVED)
            return carry
        lax.fori_loop(0, S, row_body, 0)

    fire_gather(0, rows[0], gsems[0])
    fire_gather(1, rows[1], gsems[1])

    for b in (0, 1):
        drain_gather(rows[b], gsems[b])
        compute_stage(b, rows[b], outs[b])
        fire_out(b, outs[b], osems[b])
        fire_gather(b + 2, rows[b], gsems[b])

    def pair_body(p, carry):
        for b in (0, 1):
            s = 2 * p + b
            drain_gather(rows[b], gsems[b])
            drain_out(outs[b], osems[b])
            compute_stage(s, rows[b], outs[b])
            fire_out(s, outs[b], osems[b])
            fire_gather(s + 2, rows[b], gsems[b])
        return carry
    lax.fori_loop(1, NPAIRS - 1, pair_body, 0)

    for b in (0, 1):
        s = 2 * (NPAIRS - 1) + b
        drain_gather(rows[b], gsems[b])
        drain_out(outs[b], osems[b])
        compute_stage(s, rows[b], outs[b])
        fire_out(s, outs[b], osems[b])

    drain_out(outs[0], osems[0])
    drain_out(outs[1], osems[1])


def kernel(input_ids, mask, table):
    ids2d = input_ids.reshape(N // G, G).astype(jnp.int32)
    maskf = mask.reshape(N).astype(jnp.float32)
    packed = _sc_encode(ids2d, maskf, table)
    o = packed.reshape(B, L, E, 2)
    return lax.complex(o[..., 0].astype(jnp.float32),
                       o[..., 1].astype(jnp.float32))
